# Initial kernel scaffold; baseline (speedup 1.0000x reference)
import jax
import jax.numpy as jnp
from jax.experimental import pallas as pl


def kernel(indices, values, b):
    raise NotImplementedError("write your pallas kernel here")


# speedup vs baseline: 4.0447x; 4.0447x over previous
import dataclasses
import functools

import jax
import jax.numpy as jnp
from jax import lax
from jax.experimental import pallas as pl
from jax.experimental.pallas import tpu as pltpu
from jax.experimental.pallas import tpu_sc as plsc

N_NODES = 10000
N_EDGES = 320000
D_FEAT = 128

NUM_CORES = 2
NUM_SUBCORES = 16
NUM_WORKERS = NUM_CORES * NUM_SUBCORES
EDGES_PER_WORKER = N_EDGES // NUM_WORKERS
CHUNK = 80
NUM_CHUNKS = EDGES_PER_WORKER // CHUNK
N_PAD = 10240
ROWS_PER_TILE = N_PAD // NUM_SUBCORES
ZROWS = 128
LANES = 16


def _sc_compiler_params():
  cp = pltpu.CompilerParams()
  if "needs_layout_passes" in pltpu.CompilerParams.__dataclass_fields__:
    cp = dataclasses.replace(cp, needs_layout_passes=False)
  return cp


def _sc_spmm(rows, cols, vals, b):
  mesh = plsc.VectorSubcoreMesh(core_axis_name="c", subcore_axis_name="s")

  @functools.partial(
      pl.kernel,
      compiler_params=_sc_compiler_params(),
      out_type=jax.ShapeDtypeStruct((NUM_CORES * N_PAD, D_FEAT),
                                    jnp.float32),
      mesh=mesh,
      scratch_types=[
          pltpu.VMEM((CHUNK,), jnp.int32),
          pltpu.VMEM((CHUNK,), jnp.int32),
          pltpu.VMEM((CHUNK,), jnp.float32),
          pltpu.VMEM((CHUNK, D_FEAT), jnp.float32),
          pltpu.VMEM((ZROWS, D_FEAT), jnp.float32),
          pltpu.VMEM_SHARED((N_PAD, D_FEAT), jnp.float32),
          pltpu.SemaphoreType.DMA,
      ],
  )
  def kern(rows_hbm, cols_hbm, vals_hbm, b_hbm, out_hbm,
           rows_v, cols_v, vals_v, gath_v, zero_v, acc, sem):
    cid = lax.axis_index("c")
    sid = lax.axis_index("s")
    wid = sid * NUM_CORES + cid

    zeros16 = jnp.zeros((LANES,), jnp.float32)

    @pl.loop(0, ZROWS)
    def _(i):
      for k in range(D_FEAT // LANES):
        zero_v[i, pl.ds(k * LANES, LANES)] = zeros16

    @pl.loop(0, ROWS_PER_TILE // ZROWS)
    def _(j):
      pltpu.sync_copy(zero_v,
                      acc.at[pl.ds(sid * ROWS_PER_TILE + j * ZROWS, ZROWS)])

    plsc.subcore_barrier()

    @pl.loop(0, NUM_CHUNKS)
    def _(c):
      base = wid * EDGES_PER_WORKER + c * CHUNK
      pltpu.sync_copy(rows_hbm.at[pl.ds(base, CHUNK)], rows_v)
      pltpu.sync_copy(cols_hbm.at[pl.ds(base, CHUNK)], cols_v)
      pltpu.sync_copy(vals_hbm.at[pl.ds(base, CHUNK)], vals_v)
      pltpu.async_copy(b_hbm.at[cols_v], gath_v, sem).wait()

      @pl.loop(0, CHUNK)
      def _(i):
        vbc = plsc.load_gather(vals_v, [jnp.full((LANES,), i, jnp.int32)])
        for k in range(D_FEAT // LANES):
          sl = (i, pl.ds(k * LANES, LANES))
          gath_v[sl] = gath_v[sl] * vbc

      pltpu.sync_copy(gath_v, acc.at[rows_v], add=True)

    plsc.subcore_barrier()

    pltpu.sync_copy(
        acc.at[pl.ds(sid * ROWS_PER_TILE, ROWS_PER_TILE)],
        out_hbm.at[pl.ds(cid * N_PAD + sid * ROWS_PER_TILE, ROWS_PER_TILE)])

  return kern(rows, cols, vals, b)


def _add_partials(p0, p1):
  def body(a_ref, b_ref, o_ref):
    o_ref[...] = a_ref[...] + b_ref[...]

  return pl.pallas_call(
      body,
      out_shape=jax.ShapeDtypeStruct((N_NODES, D_FEAT), jnp.float32),
  )(p0, p1)


@jax.jit
def kernel(indices, values, b):
  rows = indices[0]
  cols = indices[1]
  partials = _sc_spmm(rows, cols, values, b)
  return _add_partials(partials[:N_NODES], partials[N_PAD:N_PAD + N_NODES])


# speedup vs baseline: 4.5736x; 1.1308x over previous
import dataclasses
import functools

import jax
import jax.numpy as jnp
from jax import lax
from jax.experimental import pallas as pl
from jax.experimental.pallas import tpu as pltpu
from jax.experimental.pallas import tpu_sc as plsc

N_NODES = 10000
N_EDGES = 320000
D_FEAT = 128

NUM_CORES = 2
NUM_SUBCORES = 16
NUM_WORKERS = NUM_CORES * NUM_SUBCORES
EDGES_PER_WORKER = N_EDGES // NUM_WORKERS
CHUNK = 40
NUM_CHUNKS = EDGES_PER_WORKER // CHUNK
N_PAD = 10240
ROWS_PER_TILE = N_PAD // NUM_SUBCORES
LANES = 16


def _sc_compiler_params():
  cp = pltpu.CompilerParams()
  if "needs_layout_passes" in pltpu.CompilerParams.__dataclass_fields__:
    cp = dataclasses.replace(cp, needs_layout_passes=False)
  return cp


def _sc_spmm(rows, cols, vals, b):
  mesh = plsc.VectorSubcoreMesh(core_axis_name="c", subcore_axis_name="s")

  @functools.partial(
      pl.kernel,
      compiler_params=_sc_compiler_params(),
      out_type=jax.ShapeDtypeStruct((NUM_CORES * N_PAD, D_FEAT),
                                    jnp.float32),
      mesh=mesh,
      scratch_types=[
          pltpu.VMEM((EDGES_PER_WORKER,), jnp.int32),
          pltpu.VMEM((CHUNK,), jnp.int32),
          pltpu.VMEM((CHUNK,), jnp.int32),
          pltpu.VMEM((CHUNK,), jnp.int32),
          pltpu.VMEM((CHUNK,), jnp.int32),
          pltpu.VMEM((CHUNK,), jnp.float32),
          pltpu.VMEM((CHUNK,), jnp.float32),
          pltpu.VMEM((CHUNK, D_FEAT), jnp.float32),
          pltpu.VMEM((CHUNK, D_FEAT), jnp.float32),
          pltpu.VMEM((CHUNK, D_FEAT), jnp.float32),
          pltpu.VMEM((CHUNK, D_FEAT), jnp.float32),
          pltpu.VMEM_SHARED((N_PAD, D_FEAT), jnp.float32),
          pltpu.SemaphoreType.DMA,
          pltpu.SemaphoreType.DMA,
          pltpu.SemaphoreType.DMA,
          pltpu.SemaphoreType.DMA,
      ],
  )
  def kern(rows_hbm, cols_hbm, vals_hbm, b_hbm, out_hbm,
           cols_v, r0, r1, si0, si1, v0, v1, g0, g1, s0, s1, acc,
           gsem0, gsem1, ssem0, ssem1):
    cid = lax.axis_index("c")
    sid = lax.axis_index("s")
    wid = sid * NUM_CORES + cid
    rbuf = (r0, r1)
    sibuf = (si0, si1)
    vbuf = (v0, v1)
    gbuf = (g0, g1)
    sbuf = (s0, s1)
    gsem = (gsem0, gsem1)
    ssem = (ssem0, ssem1)

    pltpu.sync_copy(cols_hbm.at[wid], cols_v)

    zeros16 = jnp.zeros((LANES,), jnp.float32)

    @pl.loop(0, CHUNK)
    def _(i):
      for k in range(D_FEAT // LANES):
        s0[i, pl.ds(k * LANES, LANES)] = zeros16

    @pl.loop(0, ROWS_PER_TILE // CHUNK)
    def _(j):
      pltpu.sync_copy(s0,
                      acc.at[pl.ds(sid * ROWS_PER_TILE + j * CHUNK, CHUNK)])

    plsc.subcore_barrier()

    def fire(c, p):
      base = wid * EDGES_PER_WORKER + c * CHUNK
      pltpu.async_copy(rows_hbm.at[pl.ds(base, CHUNK)], rbuf[p], gsem[p])
      pltpu.async_copy(vals_hbm.at[pl.ds(base, CHUNK)], vbuf[p], gsem[p])
      pltpu.async_copy(
          b_hbm.at[cols_v.at[pl.ds(c * CHUNK, CHUNK)]], gbuf[p], gsem[p])

    def wait_fire(p):
      base0 = wid * EDGES_PER_WORKER
      pltpu.make_async_copy(rows_hbm.at[pl.ds(base0, CHUNK)],
                            rbuf[p], gsem[p]).wait()
      pltpu.make_async_copy(vals_hbm.at[pl.ds(base0, CHUNK)],
                            vbuf[p], gsem[p]).wait()
      pltpu.make_async_copy(b_hbm.at[cols_v.at[pl.ds(0, CHUNK)]],
                            gbuf[p], gsem[p]).wait()

    def drain_scatter(p):
      pltpu.make_async_copy(sbuf[p], acc.at[sibuf[p]], ssem[p]).wait()

    def scale(p):
      @pl.loop(0, CHUNK)
      def _(i):
        vbc = plsc.load_gather(vbuf[p], [jnp.full((LANES,), i, jnp.int32)])
        for k in range(D_FEAT // LANES):
          sl = (i, pl.ds(k * LANES, LANES))
          sbuf[p][sl] = gbuf[p][sl] * vbc

    fire(0, 0)
    fire(1, 1)

    @pl.loop(0, NUM_CHUNKS, step=2)
    def _(c0):
      for p in range(2):
        c = c0 + p
        @pl.when(c0 >= 2)
        def _():
          drain_scatter(p)
        wait_fire(p)
        scale(p)
        for off in (0, 16, 24):
          sibuf[p][pl.ds(off, LANES)] = rbuf[p][pl.ds(off, LANES)]
        pltpu.async_copy(sbuf[p], acc.at[sibuf[p]], ssem[p], add=True)
        @pl.when(c + 2 < NUM_CHUNKS)
        def _():
          fire(c + 2, p)

    for p in range(2):
      drain_scatter(p)

    plsc.subcore_barrier()

    pltpu.sync_copy(
        acc.at[pl.ds(sid * ROWS_PER_TILE, ROWS_PER_TILE)],
        out_hbm.at[pl.ds(cid * N_PAD + sid * ROWS_PER_TILE, ROWS_PER_TILE)])

  return kern(rows, cols, vals, b)


def _add_partials(p0, p1):
  def body(a_ref, b_ref, o_ref):
    o_ref[...] = a_ref[...] + b_ref[...]

  return pl.pallas_call(
      body,
      out_shape=jax.ShapeDtypeStruct((N_NODES, D_FEAT), jnp.float32),
  )(p0, p1)


@jax.jit
def kernel(indices, values, b):
  rows = indices[0]
  cols = indices[1].reshape(NUM_WORKERS, EDGES_PER_WORKER)
  partials = _sc_spmm(rows, cols, values, b)
  return _add_partials(partials[:N_NODES], partials[N_PAD:N_PAD + N_NODES])


# speedup vs baseline: 10.1151x; 2.2116x over previous
import dataclasses
import functools

import jax
import jax.numpy as jnp
from jax import lax
from jax.experimental import pallas as pl
from jax.experimental.pallas import tpu as pltpu
from jax.experimental.pallas import tpu_sc as plsc

N_NODES = 10000
N_EDGES = 320000
D_FEAT = 128

NUM_CORES = 2
NUM_SUBCORES = 16
NUM_WORKERS = NUM_CORES * NUM_SUBCORES
EDGES_PER_WORKER = N_EDGES // NUM_WORKERS
CHUNK = 40
NUM_CHUNKS = EDGES_PER_WORKER // CHUNK
N_PAD = 10240
ROWS_PER_TILE = N_PAD // NUM_SUBCORES
LANES = 16


def _sc_compiler_params():
  cp = pltpu.CompilerParams()
  if "needs_layout_passes" in pltpu.CompilerParams.__dataclass_fields__:
    cp = dataclasses.replace(cp, needs_layout_passes=False)
  return cp


def _sc_spmm(rows, cols, vals, b):
  mesh = plsc.VectorSubcoreMesh(core_axis_name="c", subcore_axis_name="s")

  @functools.partial(
      pl.kernel,
      compiler_params=_sc_compiler_params(),
      out_type=jax.ShapeDtypeStruct((NUM_CORES * N_PAD, D_FEAT),
                                    jnp.float32),
      mesh=mesh,
      scratch_types=[
          pltpu.VMEM((EDGES_PER_WORKER,), jnp.int32),
          pltpu.VMEM((CHUNK,), jnp.int32),
          pltpu.VMEM((CHUNK,), jnp.int32),
          pltpu.VMEM((CHUNK,), jnp.int32),
          pltpu.VMEM((CHUNK,), jnp.int32),
          pltpu.VMEM((CHUNK,), jnp.float32),
          pltpu.VMEM((CHUNK,), jnp.float32),
          pltpu.VMEM((CHUNK, D_FEAT), jnp.float32),
          pltpu.VMEM((CHUNK, D_FEAT), jnp.float32),
          pltpu.VMEM((CHUNK, D_FEAT), jnp.float32),
          pltpu.VMEM((CHUNK, D_FEAT), jnp.float32),
          pltpu.VMEM_SHARED((N_PAD, D_FEAT), jnp.float32),
          pltpu.SemaphoreType.DMA,
          pltpu.SemaphoreType.DMA,
          pltpu.SemaphoreType.DMA,
          pltpu.SemaphoreType.DMA,
      ],
  )
  def kern(rows_hbm, cols_hbm, vals_hbm, b_hbm, out_hbm,
           cols_v, r0, r1, si0, si1, v0, v1, g0, g1, s0, s1, acc,
           gsem0, gsem1, ssem0, ssem1):
    cid = lax.axis_index("c")
    sid = lax.axis_index("s")
    wid = sid * NUM_CORES + cid
    rbuf = (r0, r1)
    sibuf = (si0, si1)
    vbuf = (v0, v1)
    gbuf = (g0, g1)
    sbuf = (s0, s1)
    gsem = (gsem0, gsem1)
    ssem = (ssem0, ssem1)

    pltpu.sync_copy(cols_hbm.at[wid], cols_v)

    zeros16 = jnp.zeros((LANES,), jnp.float32)

    @pl.loop(0, CHUNK)
    def _(i):
      for k in range(D_FEAT // LANES):
        s0[i, pl.ds(k * LANES, LANES)] = zeros16

    @pl.loop(0, ROWS_PER_TILE // CHUNK)
    def _(j):
      pltpu.sync_copy(s0,
                      acc.at[pl.ds(sid * ROWS_PER_TILE + j * CHUNK, CHUNK)])

    plsc.subcore_barrier()

    def fire(c, p):
      base = wid * EDGES_PER_WORKER + c * CHUNK
      pltpu.async_copy(rows_hbm.at[pl.ds(base, CHUNK)], rbuf[p], gsem[p])
      pltpu.async_copy(vals_hbm.at[pl.ds(base, CHUNK)], vbuf[p], gsem[p])
      pltpu.async_copy(
          b_hbm.at[cols_v.at[pl.ds(c * CHUNK, CHUNK)]], gbuf[p], gsem[p])

    def wait_fire(p):
      base0 = wid * EDGES_PER_WORKER
      pltpu.make_async_copy(rows_hbm.at[pl.ds(base0, CHUNK)],
                            rbuf[p], gsem[p]).wait()
      pltpu.make_async_copy(vals_hbm.at[pl.ds(base0, CHUNK)],
                            vbuf[p], gsem[p]).wait()
      pltpu.make_async_copy(b_hbm.at[cols_v.at[pl.ds(0, CHUNK)]],
                            gbuf[p], gsem[p]).wait()

    def drain_scatter(p):
      pltpu.make_async_copy(sbuf[p], acc.at[sibuf[p]], ssem[p]).wait()

    def scale(p):
      @plsc.parallel_loop(0, CHUNK, unroll=4)
      def _(i):
        vbc = plsc.load_gather(vbuf[p], [jnp.full((LANES,), i, jnp.int32)])
        for k in range(D_FEAT // LANES):
          sl = (i, pl.ds(k * LANES, LANES))
          sbuf[p][sl] = gbuf[p][sl] * vbc

    fire(0, 0)
    fire(1, 1)

    @pl.loop(0, NUM_CHUNKS, step=2)
    def _(c0):
      for p in range(2):
        c = c0 + p
        @pl.when(c0 >= 2)
        def _():
          drain_scatter(p)
        wait_fire(p)
        scale(p)
        for off in (0, 16, 24):
          sibuf[p][pl.ds(off, LANES)] = rbuf[p][pl.ds(off, LANES)]
        pltpu.async_copy(sbuf[p], acc.at[sibuf[p]], ssem[p], add=True)
        @pl.when(c + 2 < NUM_CHUNKS)
        def _():
          fire(c + 2, p)

    for p in range(2):
      drain_scatter(p)

    plsc.subcore_barrier()

    pltpu.sync_copy(
        acc.at[pl.ds(sid * ROWS_PER_TILE, ROWS_PER_TILE)],
        out_hbm.at[pl.ds(cid * N_PAD + sid * ROWS_PER_TILE, ROWS_PER_TILE)])

  return kern(rows, cols, vals, b)


def _add_partials(p0, p1):
  def body(a_ref, b_ref, o_ref):
    o_ref[...] = a_ref[...] + b_ref[...]

  return pl.pallas_call(
      body,
      out_shape=jax.ShapeDtypeStruct((N_NODES, D_FEAT), jnp.float32),
  )(p0, p1)


@jax.jit
def kernel(indices, values, b):
  rows = indices[0]
  cols = indices[1].reshape(NUM_WORKERS, EDGES_PER_WORKER)
  partials = _sc_spmm(rows, cols, values, b)
  return _add_partials(partials[:N_NODES], partials[N_PAD:N_PAD + N_NODES])
